# baseline (speedup 1.0000x reference)
import jax
import jax.numpy as jnp
from jax.experimental import pallas as pl
from jax.experimental.pallas import tpu as pltpu

_NUM_SLICES = 8


def _copy_body(src_ref, dst_ref, sems):
    rows = src_ref.shape[0]
    chunk = rows // _NUM_SLICES
    copies = []
    for i in range(_NUM_SLICES):
        lo = i * chunk
        hi = rows if i == _NUM_SLICES - 1 else lo + chunk
        c = pltpu.make_async_copy(
            src_ref.at[pl.ds(lo, hi - lo), :],
            dst_ref.at[pl.ds(lo, hi - lo), :],
            sems.at[i],
        )
        c.start()
        copies.append(c)
    for c in copies:
        c.wait()


def kernel(inputs, ent_embeds):
    del inputs
    return pl.pallas_call(
        _copy_body,
        out_shape=jax.ShapeDtypeStruct(ent_embeds.shape, ent_embeds.dtype),
        in_specs=[pl.BlockSpec(memory_space=pltpu.MemorySpace.HBM)],
        out_specs=pl.BlockSpec(memory_space=pltpu.MemorySpace.HBM),
        scratch_shapes=[pltpu.SemaphoreType.DMA((_NUM_SLICES,))],
    )(ent_embeds)


# speedup vs baseline: 16.0980x; 16.0980x over previous
import jax
import jax.numpy as jnp
from jax.experimental import pallas as pl
from jax.experimental.pallas import tpu as pltpu

_BLOCK_ROWS = 25000


def _copy_body(src_ref, dst_ref):
    dst_ref[...] = src_ref[...]


def kernel(inputs, ent_embeds):
    del inputs
    rows, dim = ent_embeds.shape
    grid = rows // _BLOCK_ROWS
    return pl.pallas_call(
        _copy_body,
        out_shape=jax.ShapeDtypeStruct(ent_embeds.shape, ent_embeds.dtype),
        grid=(grid,),
        in_specs=[pl.BlockSpec((_BLOCK_ROWS, dim), lambda i: (i, 0))],
        out_specs=pl.BlockSpec((_BLOCK_ROWS, dim), lambda i: (i, 0)),
    )(ent_embeds)


# speedup vs baseline: 16.1148x; 1.0010x over previous
import jax
import jax.numpy as jnp
from jax.experimental import pallas as pl
from jax.experimental.pallas import tpu as pltpu

_CHUNK_ROWS = 10000
_DEPTH = 4
_SLOTS = 2 * _DEPTH


def _copy_body(src_ref, dst_ref, bufs, in_sems, out_sems):
    rows = src_ref.shape[0]
    nchunks = rows // _CHUNK_ROWS

    def in_copy(c, slot):
        return pltpu.make_async_copy(
            src_ref.at[pl.ds(c * _CHUNK_ROWS, _CHUNK_ROWS), :],
            bufs.at[slot],
            in_sems.at[slot],
        )

    def out_copy(c, slot):
        return pltpu.make_async_copy(
            bufs.at[slot],
            dst_ref.at[pl.ds(c * _CHUNK_ROWS, _CHUNK_ROWS), :],
            out_sems.at[slot],
        )

    for c in range(_DEPTH):
        in_copy(c, c % _SLOTS).start()

    for i in range(nchunks):
        slot = i % _SLOTS
        in_copy(i, slot).wait()
        out_copy(i, slot).start()
        nxt = i + _DEPTH
        if nxt < nchunks:
            nslot = nxt % _SLOTS
            if nxt >= _SLOTS:
                out_copy(nxt - _SLOTS, nslot).wait()
            in_copy(nxt, nslot).start()

    for k in range(min(_SLOTS, nchunks)):
        c = nchunks - min(_SLOTS, nchunks) + k
        out_copy(c, c % _SLOTS).wait()


def kernel(inputs, ent_embeds):
    del inputs
    rows, dim = ent_embeds.shape
    return pl.pallas_call(
        _copy_body,
        out_shape=jax.ShapeDtypeStruct(ent_embeds.shape, ent_embeds.dtype),
        in_specs=[pl.BlockSpec(memory_space=pltpu.MemorySpace.HBM)],
        out_specs=pl.BlockSpec(memory_space=pltpu.MemorySpace.HBM),
        scratch_shapes=[
            pltpu.VMEM((_SLOTS, _CHUNK_ROWS, 64), jnp.float32),
            pltpu.SemaphoreType.DMA((_SLOTS,)),
            pltpu.SemaphoreType.DMA((_SLOTS,)),
        ],
    )(ent_embeds)
